# Initial kernel scaffold; baseline (speedup 1.0000x reference)
import jax
import jax.numpy as jnp
from jax.experimental import pallas as pl


def kernel(center, pos_c, neg_c, center_table, context_table):
    raise NotImplementedError("write your pallas kernel here")


# speedup vs baseline: 1.5777x; 1.5777x over previous
import functools

import jax
import jax.numpy as jnp
from jax import lax
from jax.experimental import pallas as pl
from jax.experimental.pallas import tpu as pltpu
from jax.experimental.pallas import tpu_sc as plsc

VOCAB = 1000000
EMBED = 64
BATCH = 16384
NEG_K = 5

_info = plsc.get_sparse_core_info()
_NC = _info.num_cores
_NS = _info.num_subcores
_L = _info.num_lanes
_NW = _NC * _NS
_BPW = BATCH // _NW
_CH = 128
_NCH = _BPW // _CH
_NG = _CH // _L


def _sc_dots(center, pos_c, neg_t, center_table, context_table):
  mesh = plsc.VectorSubcoreMesh(core_axis_name="c", subcore_axis_name="s")

  @functools.partial(
      pl.kernel,
      out_type=(
          jax.ShapeDtypeStruct((BATCH,), jnp.float32),
          jax.ShapeDtypeStruct((NEG_K * BATCH,), jnp.float32),
      ),
      mesh=mesh,
      scratch_types=[
          pltpu.VMEM((_CH,), jnp.int32),
          pltpu.VMEM((_CH,), jnp.int32),
          pltpu.VMEM((NEG_K, _CH), jnp.int32),
          pltpu.VMEM((_CH, EMBED), jnp.float32),
          pltpu.VMEM((_CH, EMBED), jnp.float32),
          pltpu.VMEM((NEG_K, _CH, EMBED), jnp.float32),
          pltpu.VMEM((_CH,), jnp.float32),
          pltpu.VMEM((NEG_K, _CH), jnp.float32),
          pltpu.SemaphoreType.DMA,
      ],
      compiler_params=pltpu.CompilerParams(use_tc_tiling_on_sc=False,
                                           needs_layout_passes=False),
  )
  def k(center_hbm, pos_hbm, negt_hbm, ctab_hbm, xtab_hbm,
        pout_hbm, nout_hbm,
        cidx, pidx, nidx, cbuf, pbuf, nbuf, opos, oneg, sem):
    wid = lax.axis_index("s") * _NC + lax.axis_index("c")
    lanes = lax.iota(jnp.int32, _L)

    for ci in range(_NCH):
      base = wid * _BPW + ci * _CH
      pltpu.sync_copy(center_hbm.at[pl.ds(base, _CH)], cidx)
      pltpu.sync_copy(pos_hbm.at[pl.ds(base, _CH)], pidx)
      for kk in range(NEG_K):
        pltpu.sync_copy(negt_hbm.at[pl.ds(kk * BATCH + base, _CH)],
                        nidx.at[kk])

      copies = [
          pltpu.async_copy(ctab_hbm.at[cidx], cbuf, sem),
          pltpu.async_copy(xtab_hbm.at[pidx], pbuf, sem),
      ]
      for kk in range(NEG_K):
        copies.append(
            pltpu.async_copy(xtab_hbm.at[nidx.at[kk]], nbuf.at[kk], sem))
      for cp in copies:
        cp.wait()

      def group(g, _):
        rows = g * _L + lanes

        def dstep(d, accs):
          col = jnp.full((_L,), d, jnp.int32)
          c = plsc.load_gather(cbuf, [rows, col])
          p = plsc.load_gather(pbuf, [rows, col])
          new = [accs[0] + c * p]
          for kk in range(NEG_K):
            n = plsc.load_gather(nbuf.at[kk], [rows, col])
            new.append(accs[kk + 1] + c * n)
          return tuple(new)

        accs = lax.fori_loop(
            0, EMBED, dstep,
            tuple(jnp.zeros((_L,), jnp.float32) for _ in range(1 + NEG_K)))
        opos[pl.ds(g * _L, _L)] = accs[0]
        for kk in range(NEG_K):
          oneg[kk, pl.ds(g * _L, _L)] = accs[kk + 1]
        return 0

      lax.fori_loop(0, _NG, group, 0)

      pltpu.sync_copy(opos, pout_hbm.at[pl.ds(base, _CH)])
      for kk in range(NEG_K):
        pltpu.sync_copy(oneg.at[kk],
                        nout_hbm.at[pl.ds(kk * BATCH + base, _CH)])

  return k(center, pos_c, neg_t, center_table, context_table)


def _tc_loss(pos_dots, neg_dots):
  def body(p_ref, n_ref, o_ref):
    s = jnp.clip(p_ref[...], -10.0, 10.0)
    pos_term = jnp.log(1.0 + jnp.exp(-s))
    ns = jnp.clip(n_ref[...], -10.0, 10.0)
    neg_term = 1.0 / (1.0 + jnp.exp(ns))
    o_ref[0, 0] = (jnp.sum(pos_term) - jnp.sum(neg_term)) / BATCH

  out = pl.pallas_call(
      body,
      out_shape=jax.ShapeDtypeStruct((1, 1), jnp.float32),
      out_specs=pl.BlockSpec(memory_space=pltpu.SMEM),
  )(pos_dots, neg_dots)
  return out[0, 0]


def kernel(center, pos_c, neg_c, center_table, context_table):
  center = center.astype(jnp.int32)
  pos_c = pos_c.astype(jnp.int32)
  neg_t = jnp.transpose(neg_c.astype(jnp.int32)).reshape(-1)
  pos_dots, neg_dots = _sc_dots(center, pos_c, neg_t,
                                center_table, context_table)
  return _tc_loss(pos_dots.reshape(128, 128),
                  neg_dots.reshape(NEG_K * 128, 128))
